# baseline (speedup 1.0000x reference)
import functools

import jax
import jax.numpy as jnp
import numpy as np
from jax import lax
from jax.experimental import pallas as pl
from jax.experimental.pallas import tpu as pltpu
from jax.experimental.pallas import tpu_sc as plsc

_NC = 2
_NS = 16
_NW = _NC * _NS
_CHUNK = 128
_L = 16


def _round_up(v, m):
  return (v + m - 1) // m * m


def _unpack_perm(dh):
  perm = []
  for k in range(dh // (2 * _L)):
    base = 2 * _L * k
    perm.extend(base + 2 * j for j in range(_L))
    perm.extend(base + 2 * j + 1 for j in range(_L))
  return np.array(perm, dtype=np.int32)


def _make_sc_segment_sum(npad, dh, cpt, with_counts):
  rows_per_sub = npad // _NS
  zero_chunks = rows_per_sub // _CHUNK
  ngrp = dh // (2 * _L)
  nbuf = 2
  mesh = plsc.VectorSubcoreMesh(core_axis_name="c", subcore_axis_name="s")
  out_type = [jax.ShapeDtypeStruct((_NC, npad, dh), jnp.float32)]
  if with_counts:
    out_type.append(jax.ShapeDtypeStruct((_NC, npad, 8), jnp.float32))
  scratch = (
      [pltpu.VMEM((cpt, _CHUNK), jnp.int32),
       pltpu.VMEM((cpt, _CHUNK), jnp.int32)]
      + [pltpu.VMEM((_CHUNK, dh), jnp.bfloat16) for _ in range(nbuf)]
      + [pltpu.VMEM((_CHUNK, dh), jnp.float32),
         pltpu.VMEM((_CHUNK, dh), jnp.float32),
         pltpu.VMEM((2 * _CHUNK, 8), jnp.float32),
         pltpu.VMEM_SHARED((npad, dh), jnp.float32),
         pltpu.VMEM_SHARED((npad, 8), jnp.float32)]
      + [pltpu.SemaphoreType.DMA for _ in range(nbuf)]
  )

  def body(*refs):
    x_h, src_h, dst_h, zrow_h, w8_h = refs[:5]
    sums_o = refs[5]
    cnt_o = refs[6] if with_counts else None
    sref = refs[6 + (1 if with_counts else 0):]
    srcv, dstv = sref[0], sref[1]
    rbufs = sref[2:2 + nbuf]
    fbuf, zbuf, w8v, sums_sh, cnt_sh = sref[2 + nbuf:7 + nbuf]
    sems = sref[7 + nbuf:7 + 2 * nbuf]
    c = lax.axis_index("c")
    s = lax.axis_index("s")
    w = c * _NS + s

    pltpu.sync_copy(src_h.at[w], srcv)
    pltpu.sync_copy(dst_h.at[w], dstv)
    pltpu.sync_copy(zrow_h, zbuf)
    pltpu.sync_copy(w8_h, w8v)

    base = s * rows_per_sub
    for k in range(zero_chunks):
      off = base + k * _CHUNK
      pltpu.sync_copy(zbuf, sums_sh.at[pl.ds(off, _CHUNK)])
      if with_counts:
        pltpu.sync_copy(w8v.at[pl.ds(_CHUNK, _CHUNK)],
                        cnt_sh.at[pl.ds(off, _CHUNK)])
    plsc.subcore_barrier()

    for b in range(nbuf):
      pltpu.async_copy(x_h.at[srcv.at[b]], rbufs[b], sems[b])

    def convert(rows):
      def row(i, carry):
        for k in range(ngrp):
          v = rows[i, pl.ds(2 * _L * k, 2 * _L)]
          wv = plsc.bitcast(v, jnp.int32)
          fbuf[i, pl.ds(2 * _L * k, _L)] = plsc.bitcast(
              wv << 16, jnp.float32)
          fbuf[i, pl.ds(2 * _L * k + _L, _L)] = plsc.bitcast(
              wv & jnp.int32(-65536), jnp.float32)
        return carry
      lax.fori_loop(0, _CHUNK, row, 0)

    def step(j, rows, sem, issue_next):
      pltpu.make_async_copy(x_h.at[srcv.at[j]], rows, sem).wait()
      convert(rows)
      if issue_next:
        pltpu.async_copy(x_h.at[srcv.at[j + nbuf]], rows, sem)
      pltpu.sync_copy(fbuf, sums_sh.at[dstv.at[j]], add=True)
      if with_counts:
        pltpu.sync_copy(w8v.at[pl.ds(0, _CHUNK)],
                        cnt_sh.at[dstv.at[j]], add=True)

    def group(i, carry):
      for b in range(nbuf):
        step(nbuf * i + b, rbufs[b], sems[b], True)
      return carry

    lax.fori_loop(0, cpt // nbuf - 1, group, 0)
    for b in range(nbuf):
      step(cpt - nbuf + b, rbufs[b], sems[b], False)

    plsc.subcore_barrier()
    for k in range(zero_chunks):
      off = base + k * _CHUNK
      pltpu.sync_copy(sums_sh.at[pl.ds(off, _CHUNK)],
                      sums_o.at[c, pl.ds(off, _CHUNK)])
      if with_counts:
        pltpu.sync_copy(cnt_sh.at[pl.ds(off, _CHUNK)],
                        cnt_o.at[c, pl.ds(off, _CHUNK)])

  return pl.kernel(body, out_type=tuple(out_type), mesh=mesh,
                   scratch_types=scratch,
                   compiler_params=pltpu.CompilerParams(
                       use_tc_tiling_on_sc=False,
                       needs_layout_passes=False))


def _combine_body(s_ref, c8_ref, x_ref, wl_ref, bl_ref, wr_ref, o_ref,
                  obf_ref=None, *, dh, relu, split_out):
  cnt = jnp.maximum(c8_ref[0, :, 0:1], 1.0)
  dn = (((1,), (0,)), ((), ()))
  mm = functools.partial(lax.dot_general, dimension_numbers=dn,
                         precision=lax.Precision.HIGHEST,
                         preferred_element_type=jnp.float32)
  r = bl_ref[0:1, :]
  for c in range(_NC):
    agg = s_ref[c] / cnt
    r = r + mm(agg, wl_ref[c * dh:(c + 1) * dh, :])
    r = r + mm(x_ref[c], wr_ref[c * dh:(c + 1) * dh, :])
  if relu:
    r = jnp.maximum(r, 0.0)
  if split_out:
    for c in range(_NC):
      half = r[:, c * dh:(c + 1) * dh]
      o_ref[c] = half
      obf_ref[c] = half.astype(jnp.bfloat16)
  else:
    o_ref[...] = r


def _make_combine(npad, d, dh, bn, relu, split_out):
  if split_out:
    out_shape = [jax.ShapeDtypeStruct((_NC, npad, dh), jnp.float32),
                 jax.ShapeDtypeStruct((_NC, npad, dh), jnp.bfloat16)]
    out_spec = [pl.BlockSpec((_NC, bn, dh), lambda i: (0, i, 0)),
                pl.BlockSpec((_NC, bn, dh), lambda i: (0, i, 0))]
  else:
    out_shape = [jax.ShapeDtypeStruct((npad, d), jnp.float32)]
    out_spec = [pl.BlockSpec((bn, d), lambda i: (i, 0))]
  return pl.pallas_call(
      functools.partial(_combine_body, dh=dh, relu=relu, split_out=split_out),
      grid=(npad // bn,),
      in_specs=[
          pl.BlockSpec((_NC, bn, dh), lambda i: (0, i, 0)),
          pl.BlockSpec((1, bn, 8), lambda i: (0, i, 0)),
          pl.BlockSpec((_NC, bn, dh), lambda i: (0, i, 0)),
          pl.BlockSpec((d, d), lambda i: (0, 0)),
          pl.BlockSpec((8, d), lambda i: (0, 0)),
          pl.BlockSpec((d, d), lambda i: (0, 0)),
      ],
      out_specs=out_spec,
      out_shape=out_shape,
  )


def kernel(x, edge_index, Wl1, bl1, Wr1, Wl2, bl2, Wr2):
  n, d = x.shape
  dh = d // _NC
  e = edge_index.shape[1]

  cpt = _round_up(-(-e // (_NS * _CHUNK)), 2)
  ep = _NS * cpt * _CHUNK
  npad = _round_up(n + 1, _NS * _CHUNK)
  bn = 1024

  src = edge_index[0].astype(jnp.int32)
  dst = edge_index[1].astype(jnp.int32)
  pad = ep - e
  src_s = jnp.concatenate(
      [src, jnp.zeros((pad,), jnp.int32)]).reshape(_NS, cpt, _CHUNK)
  src3 = jnp.concatenate([src_s, src_s + npad], axis=0)
  dst_s = jnp.concatenate(
      [dst, jnp.full((pad,), n, jnp.int32)]).reshape(_NS, cpt, _CHUNK)
  dst3 = jnp.concatenate([dst_s, dst_s], axis=0)

  xp = jnp.pad(x, ((0, npad - n), (0, 0)))
  x2 = jnp.stack([xp[:, :dh], xp[:, dh:]])
  x2bf = x2.astype(jnp.bfloat16)

  perm = _unpack_perm(dh)
  mrows = jnp.asarray(np.concatenate([c * dh + perm for c in range(_NC)]))
  wl1p = Wl1[mrows, :]
  wl2p = Wl2[mrows, :]

  zrow = jnp.zeros((_CHUNK, dh), jnp.float32)
  w8 = jnp.concatenate([jnp.ones((_CHUNK, 8), jnp.float32),
                        jnp.zeros((_CHUNK, 8), jnp.float32)], axis=0)
  bl1t = jnp.tile(bl1[None, :], (8, 1))
  bl2t = jnp.tile(bl2[None, :], (8, 1))

  sc1 = _make_sc_segment_sum(npad, dh, cpt, True)
  sums1, cnt8 = sc1(x2bf.reshape(_NC * npad, dh), src3, dst3, zrow, w8)
  h2, h2bf = _make_combine(npad, d, dh, bn, True, True)(
      sums1, cnt8, x2, wl1p, bl1t, Wr1)
  sc2 = _make_sc_segment_sum(npad, dh, cpt, False)
  sums2 = sc2(h2bf.reshape(_NC * npad, dh), src3, dst3, zrow, w8)
  if isinstance(sums2, (tuple, list)):
    sums2 = sums2[0]
  (out_p,) = _make_combine(npad, d, dh, bn, False, False)(
      sums2, cnt8, h2, wl2p, bl2t, Wr2)
  return out_p[:n]


# speedup vs baseline: 1.3319x; 1.3319x over previous
import functools

import jax
import jax.numpy as jnp
from jax import lax
from jax.experimental import pallas as pl
from jax.experimental.pallas import tpu as pltpu
from jax.experimental.pallas import tpu_sc as plsc

_NC = 2
_NS = 16
_NW = _NC * _NS
_CHUNK = 128


def _round_up(v, m):
  return (v + m - 1) // m * m


def _make_sc_segment_sum(npad, dh, cpt, with_counts):
  rows_per_sub = npad // _NS
  zero_chunks = rows_per_sub // _CHUNK
  nbuf = 2
  mesh = plsc.VectorSubcoreMesh(core_axis_name="c", subcore_axis_name="s")
  out_type = [jax.ShapeDtypeStruct((_NC, npad, dh), jnp.float32)]
  if with_counts:
    out_type.append(jax.ShapeDtypeStruct((_NC, npad, 8), jnp.float32))
  scratch = (
      [pltpu.VMEM((cpt, _CHUNK), jnp.int32),
       pltpu.VMEM((cpt, _CHUNK), jnp.int32)]
      + [pltpu.VMEM((_CHUNK, dh), jnp.float32) for _ in range(nbuf)]
      + [pltpu.VMEM((_CHUNK, dh), jnp.float32),
         pltpu.VMEM((2 * _CHUNK, 8), jnp.float32),
         pltpu.VMEM_SHARED((npad, dh), jnp.float32),
         pltpu.VMEM_SHARED((npad, 8), jnp.float32)]
      + [pltpu.SemaphoreType.DMA for _ in range(nbuf)]
  )

  def body(*refs):
    x_h, src_h, dst_h, zrow_h, w8_h = refs[:5]
    sums_o = refs[5]
    cnt_o = refs[6] if with_counts else None
    sref = refs[6 + (1 if with_counts else 0):]
    srcv, dstv = sref[0], sref[1]
    rbufs = sref[2:2 + nbuf]
    zbuf, w8v, sums_sh, cnt_sh = sref[2 + nbuf:6 + nbuf]
    sems = sref[6 + nbuf:6 + 2 * nbuf]
    c = lax.axis_index("c")
    s = lax.axis_index("s")
    w = c * _NS + s

    pltpu.sync_copy(src_h.at[w], srcv)
    pltpu.sync_copy(dst_h.at[w], dstv)
    pltpu.sync_copy(zrow_h, zbuf)
    pltpu.sync_copy(w8_h, w8v)

    base = s * rows_per_sub
    for k in range(zero_chunks):
      off = base + k * _CHUNK
      pltpu.sync_copy(zbuf, sums_sh.at[pl.ds(off, _CHUNK)])
      if with_counts:
        pltpu.sync_copy(w8v.at[pl.ds(_CHUNK, _CHUNK)],
                        cnt_sh.at[pl.ds(off, _CHUNK)])
    plsc.subcore_barrier()

    for b in range(nbuf):
      pltpu.async_copy(x_h.at[srcv.at[b]], rbufs[b], sems[b])

    def step(j, rows, sem, issue_next):
      pltpu.make_async_copy(x_h.at[srcv.at[j]], rows, sem).wait()
      pltpu.sync_copy(rows, sums_sh.at[dstv.at[j]], add=True)
      if with_counts:
        pltpu.sync_copy(w8v.at[pl.ds(0, _CHUNK)],
                        cnt_sh.at[dstv.at[j]], add=True)
      if issue_next:
        pltpu.async_copy(x_h.at[srcv.at[j + nbuf]], rows, sem)

    def group(i, carry):
      for b in range(nbuf):
        step(nbuf * i + b, rbufs[b], sems[b], True)
      return carry

    lax.fori_loop(0, cpt // nbuf - 1, group, 0)
    for b in range(nbuf):
      step(cpt - nbuf + b, rbufs[b], sems[b], False)

    plsc.subcore_barrier()
    for k in range(zero_chunks):
      off = base + k * _CHUNK
      pltpu.sync_copy(sums_sh.at[pl.ds(off, _CHUNK)],
                      sums_o.at[c, pl.ds(off, _CHUNK)])
      if with_counts:
        pltpu.sync_copy(cnt_sh.at[pl.ds(off, _CHUNK)],
                        cnt_o.at[c, pl.ds(off, _CHUNK)])

  return pl.kernel(body, out_type=tuple(out_type), mesh=mesh,
                   scratch_types=scratch,
                   compiler_params=pltpu.CompilerParams(
                       use_tc_tiling_on_sc=False))


def _combine_body(s_ref, c8_ref, x_ref, wl_ref, bl_ref, wr_ref, o_ref, *,
                  dh, relu, split_out):
  cnt = jnp.maximum(c8_ref[0, :, 0:1], 1.0)
  dn = (((1,), (0,)), ((), ()))
  mm = functools.partial(lax.dot_general, dimension_numbers=dn,
                         precision=lax.Precision.HIGHEST,
                         preferred_element_type=jnp.float32)
  r = bl_ref[0:1, :]
  for c in range(_NC):
    agg = s_ref[c] / cnt
    r = r + mm(agg, wl_ref[c * dh:(c + 1) * dh, :])
    r = r + mm(x_ref[c], wr_ref[c * dh:(c + 1) * dh, :])
  if relu:
    r = jnp.maximum(r, 0.0)
  if split_out:
    for c in range(_NC):
      o_ref[c] = r[:, c * dh:(c + 1) * dh]
  else:
    o_ref[...] = r


def _make_combine(npad, d, dh, bn, relu, split_out):
  if split_out:
    out_shape = jax.ShapeDtypeStruct((_NC, npad, dh), jnp.float32)
    out_spec = pl.BlockSpec((_NC, bn, dh), lambda i: (0, i, 0))
  else:
    out_shape = jax.ShapeDtypeStruct((npad, d), jnp.float32)
    out_spec = pl.BlockSpec((bn, d), lambda i: (i, 0))
  return pl.pallas_call(
      functools.partial(_combine_body, dh=dh, relu=relu, split_out=split_out),
      grid=(npad // bn,),
      in_specs=[
          pl.BlockSpec((_NC, bn, dh), lambda i: (0, i, 0)),
          pl.BlockSpec((1, bn, 8), lambda i: (0, i, 0)),
          pl.BlockSpec((_NC, bn, dh), lambda i: (0, i, 0)),
          pl.BlockSpec((d, d), lambda i: (0, 0)),
          pl.BlockSpec((8, d), lambda i: (0, 0)),
          pl.BlockSpec((d, d), lambda i: (0, 0)),
      ],
      out_specs=out_spec,
      out_shape=out_shape,
  )


def kernel(x, edge_index, Wl1, bl1, Wr1, Wl2, bl2, Wr2):
  n, d = x.shape
  dh = d // _NC
  e = edge_index.shape[1]

  cpt = _round_up(-(-e // (_NS * _CHUNK)), 2)
  ep = _NS * cpt * _CHUNK
  npad = _round_up(n + 1, _NS * _CHUNK)
  bn = 1024

  src = edge_index[0].astype(jnp.int32)
  dst = edge_index[1].astype(jnp.int32)
  pad = ep - e
  src_s = jnp.concatenate(
      [src, jnp.zeros((pad,), jnp.int32)]).reshape(_NS, cpt, _CHUNK)
  src3 = jnp.concatenate([src_s, src_s + npad], axis=0)
  dst_s = jnp.concatenate(
      [dst, jnp.full((pad,), n, jnp.int32)]).reshape(_NS, cpt, _CHUNK)
  dst3 = jnp.concatenate([dst_s, dst_s], axis=0)

  xp = jnp.pad(x, ((0, npad - n), (0, 0)))
  x2 = jnp.stack([xp[:, :dh], xp[:, dh:]])

  zrow = jnp.zeros((_CHUNK, dh), jnp.float32)
  w8 = jnp.concatenate([jnp.ones((_CHUNK, 8), jnp.float32),
                        jnp.zeros((_CHUNK, 8), jnp.float32)], axis=0)
  bl1t = jnp.tile(bl1[None, :], (8, 1))
  bl2t = jnp.tile(bl2[None, :], (8, 1))

  sc1 = _make_sc_segment_sum(npad, dh, cpt, True)
  sums1, cnt8 = sc1(x2.reshape(_NC * npad, dh), src3, dst3, zrow, w8)
  h2 = _make_combine(npad, d, dh, bn, True, True)(
      sums1, cnt8, x2, Wl1, bl1t, Wr1)
  sc2 = _make_sc_segment_sum(npad, dh, cpt, False)
  sums2 = sc2(h2.reshape(_NC * npad, dh), src3, dst3, zrow, w8)
  if isinstance(sums2, (tuple, list)):
    sums2 = sums2[0]
  out_p = _make_combine(npad, d, dh, bn, False, False)(
      sums2, cnt8, h2, Wl2, bl2t, Wr2)
  return out_p[:n]


# speedup vs baseline: 1.3491x; 1.0129x over previous
import functools

import jax
import jax.numpy as jnp
from jax import lax
from jax.experimental import pallas as pl
from jax.experimental.pallas import tpu as pltpu
from jax.experimental.pallas import tpu_sc as plsc

_NC = 2
_NS = 16
_NW = _NC * _NS
_CHUNK = 128


def _round_up(v, m):
  return (v + m - 1) // m * m


def _make_sc_segment_sum(npad, dh, cpt, with_counts):
  rows_per_sub = npad // _NS
  zero_chunks = rows_per_sub // _CHUNK
  nbuf = 2
  mesh = plsc.VectorSubcoreMesh(core_axis_name="c", subcore_axis_name="s")
  out_type = [jax.ShapeDtypeStruct((_NC, npad, dh), jnp.float32)]
  if with_counts:
    out_type.append(jax.ShapeDtypeStruct((_NC, npad, 8), jnp.float32))
  scratch = (
      [pltpu.VMEM((cpt, _CHUNK), jnp.int32),
       pltpu.VMEM((cpt, _CHUNK), jnp.int32)]
      + [pltpu.VMEM((_CHUNK, dh), jnp.float32) for _ in range(nbuf)]
      + [pltpu.VMEM((_CHUNK, dh), jnp.float32),
         pltpu.VMEM((2 * _CHUNK, 8), jnp.float32),
         pltpu.VMEM_SHARED((npad, dh), jnp.float32),
         pltpu.VMEM_SHARED((npad, 8), jnp.float32)]
      + [pltpu.SemaphoreType.DMA for _ in range(nbuf)]
  )

  def body(*refs):
    x_h, src_h, dst_h, zrow_h, w8_h = refs[:5]
    sums_o = refs[5]
    cnt_o = refs[6] if with_counts else None
    sref = refs[6 + (1 if with_counts else 0):]
    srcv, dstv = sref[0], sref[1]
    rbufs = sref[2:2 + nbuf]
    zbuf, w8v, sums_sh, cnt_sh = sref[2 + nbuf:6 + nbuf]
    sems = sref[6 + nbuf:6 + 2 * nbuf]
    c = lax.axis_index("c")
    s = lax.axis_index("s")
    w = c * _NS + s

    pltpu.sync_copy(src_h.at[w], srcv)
    pltpu.sync_copy(dst_h.at[w], dstv)
    pltpu.sync_copy(zrow_h, zbuf)
    pltpu.sync_copy(w8_h, w8v)

    base = s * rows_per_sub
    for k in range(zero_chunks):
      off = base + k * _CHUNK
      pltpu.sync_copy(zbuf, sums_sh.at[pl.ds(off, _CHUNK)])
      if with_counts:
        pltpu.sync_copy(w8v.at[pl.ds(_CHUNK, _CHUNK)],
                        cnt_sh.at[pl.ds(off, _CHUNK)])
    plsc.subcore_barrier()

    for b in range(nbuf):
      pltpu.async_copy(x_h.at[srcv.at[b]], rbufs[b], sems[b])

    def step(j, rows, sem, issue_next):
      pltpu.make_async_copy(x_h.at[srcv.at[j]], rows, sem).wait()
      pltpu.sync_copy(rows, sums_sh.at[dstv.at[j]], add=True)
      if issue_next:
        pltpu.async_copy(x_h.at[srcv.at[j + nbuf]], rows, sem)
      if with_counts:
        pltpu.sync_copy(w8v.at[pl.ds(0, _CHUNK)],
                        cnt_sh.at[dstv.at[j]], add=True)

    def group(i, carry):
      for b in range(nbuf):
        step(nbuf * i + b, rbufs[b], sems[b], True)
      return carry

    lax.fori_loop(0, cpt // nbuf - 1, group, 0)
    for b in range(nbuf):
      step(cpt - nbuf + b, rbufs[b], sems[b], False)

    plsc.subcore_barrier()
    for k in range(zero_chunks):
      off = base + k * _CHUNK
      pltpu.sync_copy(sums_sh.at[pl.ds(off, _CHUNK)],
                      sums_o.at[c, pl.ds(off, _CHUNK)])
      if with_counts:
        pltpu.sync_copy(cnt_sh.at[pl.ds(off, _CHUNK)],
                        cnt_o.at[c, pl.ds(off, _CHUNK)])

  return pl.kernel(body, out_type=tuple(out_type), mesh=mesh,
                   scratch_types=scratch,
                   compiler_params=pltpu.CompilerParams(
                       use_tc_tiling_on_sc=False))


def _combine_body(s_ref, c8_ref, x_ref, wl_ref, bl_ref, wr_ref, o_ref, *,
                  dh, relu, split_out):
  cnt = jnp.maximum(c8_ref[0, :, 0:1], 1.0)
  dn = (((1,), (0,)), ((), ()))
  mm = functools.partial(lax.dot_general, dimension_numbers=dn,
                         precision=lax.Precision.HIGHEST,
                         preferred_element_type=jnp.float32)
  r = bl_ref[0:1, :]
  for c in range(_NC):
    agg = s_ref[c] / cnt
    r = r + mm(agg, wl_ref[c * dh:(c + 1) * dh, :])
    r = r + mm(x_ref[c], wr_ref[c * dh:(c + 1) * dh, :])
  if relu:
    r = jnp.maximum(r, 0.0)
  if split_out:
    for c in range(_NC):
      o_ref[c] = r[:, c * dh:(c + 1) * dh]
  else:
    o_ref[...] = r


def _make_combine(npad, d, dh, bn, relu, split_out):
  if split_out:
    out_shape = jax.ShapeDtypeStruct((_NC, npad, dh), jnp.float32)
    out_spec = pl.BlockSpec((_NC, bn, dh), lambda i: (0, i, 0))
  else:
    out_shape = jax.ShapeDtypeStruct((npad, d), jnp.float32)
    out_spec = pl.BlockSpec((bn, d), lambda i: (i, 0))
  return pl.pallas_call(
      functools.partial(_combine_body, dh=dh, relu=relu, split_out=split_out),
      grid=(npad // bn,),
      in_specs=[
          pl.BlockSpec((_NC, bn, dh), lambda i: (0, i, 0)),
          pl.BlockSpec((1, bn, 8), lambda i: (0, i, 0)),
          pl.BlockSpec((_NC, bn, dh), lambda i: (0, i, 0)),
          pl.BlockSpec((d, d), lambda i: (0, 0)),
          pl.BlockSpec((8, d), lambda i: (0, 0)),
          pl.BlockSpec((d, d), lambda i: (0, 0)),
      ],
      out_specs=out_spec,
      out_shape=out_shape,
  )


def kernel(x, edge_index, Wl1, bl1, Wr1, Wl2, bl2, Wr2):
  n, d = x.shape
  dh = d // _NC
  e = edge_index.shape[1]

  cpt = _round_up(-(-e // (_NS * _CHUNK)), 2)
  ep = _NS * cpt * _CHUNK
  npad = _round_up(n + 1, _NS * _CHUNK)
  bn = 1024

  src = edge_index[0].astype(jnp.int32)
  dst = edge_index[1].astype(jnp.int32)
  pad = ep - e
  src_s = jnp.concatenate(
      [src, jnp.zeros((pad,), jnp.int32)]).reshape(_NS, cpt, _CHUNK)
  src3 = jnp.concatenate([src_s, src_s + npad], axis=0)
  dst_s = jnp.concatenate(
      [dst, jnp.full((pad,), n, jnp.int32)]).reshape(_NS, cpt, _CHUNK)
  dst3 = jnp.concatenate([dst_s, dst_s], axis=0)

  xp = jnp.pad(x, ((0, npad - n), (0, 0)))
  x2 = jnp.stack([xp[:, :dh], xp[:, dh:]])

  zrow = jnp.zeros((_CHUNK, dh), jnp.float32)
  w8 = jnp.concatenate([jnp.ones((_CHUNK, 8), jnp.float32),
                        jnp.zeros((_CHUNK, 8), jnp.float32)], axis=0)
  bl1t = jnp.tile(bl1[None, :], (8, 1))
  bl2t = jnp.tile(bl2[None, :], (8, 1))

  sc1 = _make_sc_segment_sum(npad, dh, cpt, True)
  sums1, cnt8 = sc1(x2.reshape(_NC * npad, dh), src3, dst3, zrow, w8)
  h2 = _make_combine(npad, d, dh, bn, True, True)(
      sums1, cnt8, x2, Wl1, bl1t, Wr1)
  sc2 = _make_sc_segment_sum(npad, dh, cpt, False)
  sums2 = sc2(h2.reshape(_NC * npad, dh), src3, dst3, zrow, w8)
  if isinstance(sums2, (tuple, list)):
    sums2 = sums2[0]
  out_p = _make_combine(npad, d, dh, bn, False, False)(
      sums2, cnt8, h2, Wl2, bl2t, Wr2)
  return out_p[:n]
